# baseline (speedup 1.0000x reference)
import functools

import jax
import jax.numpy as jnp
from jax import lax
from jax.experimental import pallas as pl
from jax.experimental.pallas import tpu as pltpu
from jax.experimental.pallas import tpu_sc as plsc

R1, R2 = 16, 16
N1, N2 = 256, 256
B = 16384

NC, NS, L = 2, 16, 16
NW = NC * NS
BW = B // NW
CB = BW // 128
NPLANE = R1 * N1
PW = NPLANE // NW
PLINES = N2 * R2 // L
LSLICE = N1 * N2

_mesh = plsc.VectorSubcoreMesh(
    core_axis_name="c", subcore_axis_name="s",
    num_cores=NC, num_subcores=NS)
_params = pltpu.CompilerParams(
    needs_layout_passes=False, use_tc_tiling_on_sc=False)


@functools.partial(
    pl.kernel,
    out_type=jax.ShapeDtypeStruct((R1 * N1 * N2, R2), jnp.float32),
    mesh=_mesh,
    compiler_params=_params,
    scratch_types=[
        pltpu.VMEM((PLINES, L), jnp.float32),
        pltpu.VMEM((PLINES, L), jnp.float32),
        pltpu.VMEM((PLINES, 17), jnp.float32),
        pltpu.VMEM((PLINES, 17), jnp.float32),
        pltpu.VMEM((PLINES, L), jnp.float32),
        pltpu.VMEM((PLINES, L), jnp.float32),
        pltpu.SemaphoreType.DMA,
        pltpu.SemaphoreType.DMA,
        pltpu.SemaphoreType.DMA,
    ],
)
def _convert(tab_hbm, ctab_hbm,
             in0, in1, pd0, pd1, ot0, ot1, isem0, isem1, osem):
    w = lax.axis_index("s") * NC + lax.axis_index("c")
    p0 = w * PW
    iota = lax.iota(jnp.int32, L)

    ins = (in0, in1)
    pds = (pd0, pd1)
    ots = (ot0, ot1)
    isems = (isem0, isem1)

    def fire_in(p, i):
        pltpu.async_copy(
            tab_hbm.at[pl.ds((p0 + p) * PLINES, PLINES)], ins[i], isems[i])

    def drain_in(i):
        pltpu.make_async_copy(
            tab_hbm.at[pl.ds(0, PLINES)], ins[i], isems[i]).wait()

    def drain_out():
        pltpu.make_async_copy(
            ctab_hbm.at[pl.ds(0, PLINES)], in0, osem).wait()

    def work(p, i):
        inb, pdb = ins[i], pds[i]

        def scat(k, carry):
            for u in range(8):
                lnv = jnp.full((L,), k * 8 + u, jnp.int32)
                v = plsc.load_gather(inb, [lnv, iota])
                i1v = ((lnv >> 6) & 1) * 128 + (lnv & 7) * L + iota
                r2v = (lnv >> 7) * 8 + ((lnv >> 3) & 7)
                plsc.store_scatter(pdb, [i1v, r2v], v)
            return carry
        lax.fori_loop(0, PLINES // 8, scat, 0)

        pltpu.async_copy(
            pdb.at[pl.ds(0, PLINES), pl.ds(0, L)],
            ctab_hbm.at[pl.ds((p0 + p) * PLINES, PLINES)], osem)

    fire_in(0, 0)
    fire_in(1, 1)
    drain_in(0)
    work(0, 0)
    fire_in(2, 0)
    drain_in(1)
    work(1, 1)
    fire_in(3, 1)

    def pair(pp, carry):
        for i in range(2):
            p = 2 * pp + i
            drain_in(i)
            drain_out()
            work(p, i)
            fire_in(p + 2, i)
        return carry
    lax.fori_loop(1, PW // 2 - 1, pair, 0)

    for p in (PW - 2, PW - 1):
        drain_in(p % 2)
        drain_out()
        work(p, p % 2)
    drain_out()
    drain_out()


@functools.partial(
    pl.kernel,
    out_type=jax.ShapeDtypeStruct((R1 * 2, B * 8 // 128, 128), jnp.float32),
    mesh=_mesh,
    compiler_params=_params,
    scratch_types=[
        pltpu.VMEM((CB, 2, 128), jnp.int32),
        pltpu.VMEM((CB, 128), jnp.int32),
        pltpu.VMEM((BW, L), jnp.float32),
        pltpu.VMEM((BW, L), jnp.float32),
        pltpu.VMEM((BW, 17), jnp.float32),
        pltpu.VMEM((BW, 17), jnp.float32),
        pltpu.VMEM((BW * L // 128, 128), jnp.float32),
        pltpu.VMEM((BW * L // 128, 128), jnp.float32),
        pltpu.SemaphoreType.DMA,
        pltpu.SemaphoreType.DMA,
        pltpu.SemaphoreType.DMA,
    ],
)
def _gather(idx_hbm, ctab_hbm, out_hbm,
            pair_v, idq, rw0, rw1, pd0, pd1, ob0, ob1,
            gsem0, gsem1, osem):
    w = lax.axis_index("s") * NC + lax.axis_index("c")
    pltpu.sync_copy(idx_hbm.at[pl.ds(w * CB, CB)], pair_v)
    iota = lax.iota(jnp.int32, L)

    for c4 in range(CB):
        for ch in range(8):
            i0 = pair_v[c4, 0, pl.ds(ch * L, L)]
            i1 = pair_v[c4, 1, pl.ds(ch * L, L)]
            idq[c4, pl.ds(ch * L, L)] = i0 * N2 + i1

    rws = (rw0, rw1)
    pds = (pd0, pd1)
    obs = (ob0, ob1)
    gsems = (gsem0, gsem1)

    def fire(g):
        tslice = ctab_hbm.at[pl.ds(g * LSLICE, LSLICE)]
        rwv, sem = rws[g % 2], gsems[g % 2]
        for c4 in range(CB):
            pltpu.async_copy(
                tslice.at[idq.at[c4]], rwv.at[pl.ds(c4 * 128, 128)], sem)

    def drain_gather(g):
        pltpu.make_async_copy(
            ctab_hbm.at[pl.ds(0, BW)], rws[g % 2], gsems[g % 2]).wait()

    def pad(g):
        rwv, pdv = rws[g % 2], pds[g % 2]
        def body(k, carry):
            for u in range(8):
                r = k * 8 + u
                pdv[r, pl.ds(0, L)] = plsc.load_gather(
                    rwv, [jnp.full((L,), r, jnp.int32), iota])
            return carry
        lax.fori_loop(0, BW // 8, body, 0)

    def tr(g):
        pdv, obv = pds[g % 2], obs[g % 2]
        def body(m, carry):
            r2v = jnp.full((L,), (m // 32) * 8 + (m % 8), jnp.int32)
            rbase = ((m // 8) % 4) * 128
            for ch in range(8):
                rows = rbase + ch * L + iota
                obv[m, pl.ds(ch * L, L)] = plsc.load_gather(
                    pdv, [rows, r2v])
            return carry
        lax.fori_loop(0, 2 * CB * 8, body, 0)

    def fire_out(g):
        for t in range(2):
            pltpu.async_copy(
                obs[g % 2].at[pl.ds(t * 32, 32)],
                out_hbm.at[2 * g + t, pl.ds(w * 32, 32)], osem)

    def drain_out():
        pltpu.make_async_copy(
            out_hbm.at[0, pl.ds(0, BW * L // 128)], obs[0], osem).wait()

    fire(0)
    for g in range(R1):
        if g + 1 < R1:
            fire(g + 1)
        drain_gather(g)
        pad(g)
        if g >= 1:
            if g >= 3:
                drain_out()
            tr(g - 1)
            fire_out(g - 1)
    drain_out()
    tr(R1 - 1)
    fire_out(R1 - 1)
    drain_out()
    drain_out()


@jax.jit
def kernel(indices, TT_core):
    idx3 = indices.reshape(128, 128, 2).transpose(0, 2, 1)
    tab2 = (TT_core.reshape(R1, N1, 2, 128, 2, 8)
            .transpose(0, 1, 4, 2, 5, 3).reshape(R1 * N1 * N2, R2))
    ctab = _convert(tab2)
    out3 = _gather(idx3, ctab)
    return (out3.reshape(R1, 2, 128, 8, 128)
            .transpose(2, 4, 0, 1, 3).reshape(B, R1, R2))


# speedup vs baseline: 2.3389x; 2.3389x over previous
import functools

import jax
import jax.numpy as jnp
from jax import lax
from jax.experimental import pallas as pl
from jax.experimental.pallas import tpu as pltpu
from jax.experimental.pallas import tpu_sc as plsc

R1, R2 = 16, 16
N1, N2 = 256, 256
B = 16384

NC, NS, L = 2, 16, 16
NW = NC * NS
BW = B // NW
CB = BW // 128
NROW = 64
RSTRIDE = N1 * N2 * R2


def _build():
    mesh = plsc.VectorSubcoreMesh(
        core_axis_name="c", subcore_axis_name="s",
        num_cores=NC, num_subcores=NS)

    @functools.partial(
        pl.kernel,
        out_type=jax.ShapeDtypeStruct((R1 * 2, B * 8), jnp.float32),
        mesh=mesh,
        compiler_params=pltpu.CompilerParams(
            needs_layout_passes=False, use_tc_tiling_on_sc=False),
        scratch_types=[
            pltpu.VMEM((CB, 2, 128), jnp.int32),
            pltpu.VMEM((BW,), jnp.int32),
            pltpu.VMEM((NROW, 128), jnp.int32),
            pltpu.VMEM((NROW * 128,), jnp.float32),
            pltpu.VMEM((NROW * 128,), jnp.float32),
            pltpu.SemaphoreType.DMA,
            pltpu.SemaphoreType.DMA,
            pltpu.SemaphoreType.DMA,
        ],
    )
    def run(idx_hbm, tab_hbm, out_hbm,
            pair_v, gb_v, idq, dat0, dat1, gsem0, gsem1, osem):
        w = lax.axis_index("s") * NC + lax.axis_index("c")
        pltpu.sync_copy(idx_hbm.at[pl.ds(w * CB, CB)], pair_v)

        for c4 in range(CB):
            for ch in range(8):
                i0 = pair_v[c4, 0, pl.ds(ch * L, L)]
                i1 = pair_v[c4, 1, pl.ds(ch * L, L)]
                gb_v[pl.ds(c4 * 128 + ch * L, L)] = (
                    i0 * 4096 + (i1 >> 7) * 1024 + (i1 & 127))

        def expand(m, carry):
            c2 = (m // 32) * 2048 + (m % 8) * 128
            gb0 = ((m // 8) % 4) * 128
            for ch in range(8):
                gb = gb_v[pl.ds(gb0 + ch * L, L)]
                idq[m, pl.ds(ch * L, L)] = gb + c2
            return carry
        lax.fori_loop(0, NROW, expand, 0)

        dats = (dat0, dat1)
        gsems = (gsem0, gsem1)

        def fire(g):
            tslice = tab_hbm.at[pl.ds(g * RSTRIDE, RSTRIDE)]
            datv, sem = dats[g % 2], gsems[g % 2]
            def body(s, carry):
                pltpu.async_copy(
                    tslice.at[idq.at[s]], datv.at[pl.ds(s * 128, 128)], sem)
                return carry
            lax.fori_loop(0, NROW, body, 0)

        def drain(g, sem):
            pltpu.make_async_copy(
                tab_hbm.at[pl.ds(0, NROW * 128)], dats[g % 2], sem).wait()

        def fire_out(g):
            for t in range(2):
                pltpu.async_copy(
                    dats[g % 2].at[pl.ds(t * 4096, 4096)],
                    out_hbm.at[2 * g + t, pl.ds(w * 4096, 4096)], osem)

        fire(0)
        for g in range(1, R1):
            if g >= 2:
                drain(g, osem)
            fire(g)
            drain(g - 1, gsems[(g - 1) % 2])
            fire_out(g - 1)
        drain(R1, osem)
        drain(R1 - 1, gsems[(R1 - 1) % 2])
        fire_out(R1 - 1)
        drain(R1 + 1, osem)

    return run


_tt_gather = _build()


@jax.jit
def kernel(indices, TT_core):
    idx3 = indices.reshape(128, 128, 2).transpose(0, 2, 1)
    tab = (TT_core.reshape(R1, N1, 2, 128, 2, 8)
           .transpose(0, 1, 4, 2, 5, 3).reshape(R1 * N1 * N2 * R2))
    out3 = _tt_gather(idx3, tab)
    return (out3.reshape(R1, 2, 128, 8, 128)
            .transpose(2, 4, 0, 1, 3).reshape(B, R1, R2))
